# baseline (speedup 1.0000x reference)
import functools

import jax
import jax.numpy as jnp
from jax import lax
from jax.experimental import pallas as pl
from jax.experimental.pallas import tpu as pltpu
from jax.experimental.pallas import tpu_sc as plsc

_BIG_I = 1 << 20
_NEG_F = -3.0e38
_P_PRE = 8


def _sc_routing(logits_hbm, params_hbm, ids_hbm, ws_hbm, permcnt_hbm,
                logv, idx_v, ws_v, par_v, ids_all_v, act_v, perm_v, sem):
    cid = lax.axis_index("c")
    sid = lax.axis_index("s")

    @pl.when(cid == 0)
    def _core0():
        lane = lax.iota(jnp.int32, 16)
        cp_p = pltpu.async_copy(params_hbm, par_v, sem)
        cp_l = pltpu.async_copy(logits_hbm.at[pl.ds(4 * sid, 4)], logv, sem)
        cp_p.wait()
        cp_l.wait()
        pv = par_v[...]
        top_k = jnp.max(jnp.where(lane == 0, pv, 0))
        renorm = jnp.max(jnp.where(lane == 1, pv, 0))

        acc_ids = lane + 64
        acc_num = jnp.zeros((16,), jnp.float32)
        acc_den = jnp.full((16,), 1.0, jnp.float32)
        for tt in range(4):
            v = [logv[tt, pl.ds(16 * i, 16)] for i in range(4)]
            gl = [lane + 16 * i for i in range(4)]
            m = jnp.float32(_NEG_F)
            for i in range(4):
                m = jnp.maximum(m, jnp.max(v[i]))
            e = [jnp.exp(v[i] - m) for i in range(4)]
            z = jnp.float32(0.0)
            for i in range(4):
                z = z + jnp.sum(e[i])
            a1 = jnp.int32(_BIG_I)
            for i in range(4):
                a1 = jnp.minimum(a1, jnp.min(jnp.where(v[i] == m, gl[i],
                                                       _BIG_I)))
            u = [jnp.where(gl[i] == a1, _NEG_F, v[i]) for i in range(4)]
            m2 = jnp.float32(_NEG_F)
            for i in range(4):
                m2 = jnp.maximum(m2, jnp.max(u[i]))
            a2 = jnp.int32(_BIG_I)
            for i in range(4):
                a2 = jnp.minimum(a2, jnp.min(jnp.where(u[i] == m2, gl[i],
                                                       _BIG_I)))
            p2raw = jnp.float32(0.0)
            for i in range(4):
                p2raw = p2raw + jnp.sum(jnp.where(gl[i] == a2, e[i], 0.0))
            num1 = jnp.where(top_k >= 1, jnp.float32(1.0), 0.0)
            num2 = jnp.where(top_k >= 2, p2raw, 0.0)
            den = jnp.where(renorm != 0, num1 + num2, z)
            acc_ids = jnp.where(lane == 2 * tt, a1, acc_ids)
            acc_ids = jnp.where(lane == 2 * tt + 1, a2, acc_ids)
            acc_num = jnp.where(lane == 2 * tt, num1, acc_num)
            acc_num = jnp.where(lane == 2 * tt + 1, num2, acc_num)
            acc_den = jnp.where(lane == 2 * tt, den, acc_den)
            acc_den = jnp.where(lane == 2 * tt + 1, den, acc_den)
        idx_v[...] = acc_ids
        ws_v[...] = acc_num / acc_den
        cp_i = pltpu.async_copy(idx_v.at[pl.ds(0, 8)],
                                ids_hbm.at[pl.ds(8 * sid, 8)], sem)
        cp_w = pltpu.async_copy(ws_v.at[pl.ds(0, 8)],
                                ws_hbm.at[pl.ds(8 * sid, 8)], sem)
        cp_i.wait()
        cp_w.wait()

        plsc.subcore_barrier()

        @pl.when(sid == 0)
        def _compact():
            pltpu.sync_copy(ids_hbm, ids_all_v)
            for i in range(4):
                act_v[pl.ds(16 * i, 16)] = jnp.zeros((16,), jnp.int32)
            ones = jnp.full((16,), 1, jnp.int32)
            for i in range(8):
                idv = ids_all_v[pl.ds(16 * i, 16)]
                plsc.store_scatter(act_v, [idv], ones)
            for i in range(5):
                perm_v[pl.ds(16 * i, 16)] = jnp.zeros((16,), jnp.int32)
            base = jnp.int32(0)
            for i in range(4):
                a = act_v[pl.ds(16 * i, 16)]
                if i == 0:
                    a = jnp.where(lane >= _P_PRE, a, 0)
                incl = plsc.cumsum(a)
                pos = incl - a + base
                plsc.store_scatter(perm_v, [pos], lane + 16 * i, mask=a > 0)
                base = base + jnp.max(incl)
            perm_v[pl.ds(64, 16)] = jnp.full((16,), 1, jnp.int32) * base
            pltpu.sync_copy(perm_v, permcnt_hbm)


def _expert_ffn(x_ref, w13_ref, w2_ref, d_ff):
    h_ = jax.lax.dot_general(x_ref[:], w13_ref[0], (((1,), (1,)), ((), ())),
                             preferred_element_type=jnp.float32)
    gate = h_[:, :d_ff]
    up = h_[:, d_ff:]
    act = gate * jax.lax.logistic(gate) * up
    return jax.lax.dot_general(act, w2_ref[0], (((1,), (1,)), ((), ())),
                               preferred_element_type=jnp.float32)


def _prefix_kernel(x_ref, w13_ref, w2_ref, oa_ref, *, d_ff):
    oa_ref[0] = _expert_ffn(x_ref, w13_ref, w2_ref, d_ff)


def _ffn_kernel(pc_ref, ids_ref, ws_ref, x_ref, oa_ref, w13_ref, w2_ref,
                out_ref, *, d_ff, n_e):
    i = pl.program_id(0)
    cnt = pc_ref[n_e]

    @pl.when(i == 0)
    def _init():
        out_ref[:] = jnp.zeros_like(out_ref)

    def _combine(e, o):
        c = (jnp.where(ids_ref[:, 0:1] == e, ws_ref[:, 0:1], 0.0)
             + jnp.where(ids_ref[:, 1:2] == e, ws_ref[:, 1:2], 0.0))
        out_ref[:] += c * o

    @pl.when(i < _P_PRE)
    def _prefix():
        _combine(i, oa_ref[0])

    @pl.when((i >= _P_PRE) & (i - _P_PRE < cnt))
    def _compute():
        j = jnp.minimum(jnp.maximum(i - _P_PRE, 0), jnp.maximum(cnt - 1, 0))
        _combine(pc_ref[j], _expert_ffn(x_ref, w13_ref, w2_ref, d_ff))


def kernel(x, router_logits, w13_weight, w2_weight, top_k, renormalize):
    t, h = x.shape
    n_e = w13_weight.shape[0]
    d_ff = w13_weight.shape[1] // 2
    lane16 = jnp.arange(16, dtype=jnp.int32)
    params16 = (jnp.where(lane16 == 0, jnp.asarray(top_k, jnp.int32), 0)
                + jnp.where(lane16 == 1, jnp.asarray(renormalize, jnp.int32),
                            0))

    mesh = plsc.VectorSubcoreMesh(core_axis_name="c", subcore_axis_name="s",
                                  num_cores=1)
    routing = functools.partial(
        pl.kernel,
        mesh=mesh,
        compiler_params=pltpu.CompilerParams(needs_layout_passes=False),
        out_type=[
            jax.ShapeDtypeStruct((2 * t,), jnp.int32),
            jax.ShapeDtypeStruct((2 * t,), jnp.float32),
            jax.ShapeDtypeStruct((n_e + 16,), jnp.int32),
        ],
        scratch_types=[
            pltpu.VMEM((4, n_e), jnp.float32),
            pltpu.VMEM((16,), jnp.int32),
            pltpu.VMEM((16,), jnp.float32),
            pltpu.VMEM((16,), jnp.int32),
            pltpu.VMEM((2 * t,), jnp.int32),
            pltpu.VMEM((n_e,), jnp.int32),
            pltpu.VMEM((n_e + 16,), jnp.int32),
            pltpu.SemaphoreType.DMA,
        ],
    )(_sc_routing)
    ids_flat, ws_flat, permcnt = routing(router_logits, params16)
    ids = ids_flat.reshape(t, 2)
    ws = ws_flat.reshape(t, 2)

    oa = pl.pallas_call(
        functools.partial(_prefix_kernel, d_ff=d_ff),
        grid=(_P_PRE,),
        in_specs=[
            pl.BlockSpec((t, h), lambda e: (0, 0)),
            pl.BlockSpec((1, 2 * d_ff, h), lambda e: (e, 0, 0)),
            pl.BlockSpec((1, h, d_ff), lambda e: (e, 0, 0)),
        ],
        out_specs=pl.BlockSpec((1, t, h), lambda e: (e, 0, 0)),
        out_shape=jax.ShapeDtypeStruct((_P_PRE, t, h), jnp.float32),
        compiler_params=pltpu.CompilerParams(
            dimension_semantics=("arbitrary",),
        ),
    )(x, w13_weight, w2_weight)

    def _w_map(i, pc_ref):
        j = jnp.minimum(jnp.maximum(i - _P_PRE, 0),
                        jnp.maximum(pc_ref[n_e] - 1, 0))
        return (pc_ref[j], 0, 0)

    body = functools.partial(_ffn_kernel, d_ff=d_ff, n_e=n_e)
    grid_spec = pltpu.PrefetchScalarGridSpec(
        num_scalar_prefetch=1,
        grid=(_P_PRE + n_e,),
        in_specs=[
            pl.BlockSpec((t, 2), lambda i, p: (0, 0)),
            pl.BlockSpec((t, 2), lambda i, p: (0, 0)),
            pl.BlockSpec((t, h), lambda i, p: (0, 0)),
            pl.BlockSpec((1, t, h), lambda i, p: (jnp.minimum(i, _P_PRE - 1),
                                                  0, 0)),
            pl.BlockSpec((1, 2 * d_ff, h), _w_map),
            pl.BlockSpec((1, h, d_ff), _w_map),
        ],
        out_specs=pl.BlockSpec((t, h), lambda i, p: (0, 0)),
    )
    out = pl.pallas_call(
        body,
        grid_spec=grid_spec,
        out_shape=jax.ShapeDtypeStruct((t, h), jnp.float32),
        compiler_params=pltpu.CompilerParams(
            dimension_semantics=("arbitrary",),
        ),
    )(permcnt, ids, ws, x, oa, w13_weight, w2_weight)
    return out.astype(x.dtype)


# speedup vs baseline: 1.0520x; 1.0520x over previous
import functools

import jax
import jax.numpy as jnp
from jax import lax
from jax.experimental import pallas as pl
from jax.experimental.pallas import tpu as pltpu
from jax.experimental.pallas import tpu_sc as plsc

_BIG_I = 1 << 20
_NEG_F = -3.0e38


def _sc_routing(logits_hbm, params_hbm, ids_hbm, ws_hbm, permcnt_hbm,
                logv, idx_v, ws_v, par_v, ids_all_v, act_v, perm_v, sem):
    cid = lax.axis_index("c")
    sid = lax.axis_index("s")

    @pl.when(cid == 0)
    def _core0():
        lane = lax.iota(jnp.int32, 16)
        cp_p = pltpu.async_copy(params_hbm, par_v, sem)
        cp_l = pltpu.async_copy(logits_hbm.at[pl.ds(4 * sid, 4)], logv, sem)
        cp_p.wait()
        cp_l.wait()
        pv = par_v[...]
        top_k = jnp.max(jnp.where(lane == 0, pv, 0))
        renorm = jnp.max(jnp.where(lane == 1, pv, 0))

        acc_ids = lane + 64
        acc_num = jnp.zeros((16,), jnp.float32)
        acc_den = jnp.full((16,), 1.0, jnp.float32)
        for tt in range(4):
            v = [logv[tt, pl.ds(16 * i, 16)] for i in range(4)]
            gl = [lane + 16 * i for i in range(4)]
            m = jnp.float32(_NEG_F)
            for i in range(4):
                m = jnp.maximum(m, jnp.max(v[i]))
            e = [jnp.exp(v[i] - m) for i in range(4)]
            z = jnp.float32(0.0)
            for i in range(4):
                z = z + jnp.sum(e[i])
            a1 = jnp.int32(_BIG_I)
            for i in range(4):
                a1 = jnp.minimum(a1, jnp.min(jnp.where(v[i] == m, gl[i],
                                                       _BIG_I)))
            u = [jnp.where(gl[i] == a1, _NEG_F, v[i]) for i in range(4)]
            m2 = jnp.float32(_NEG_F)
            for i in range(4):
                m2 = jnp.maximum(m2, jnp.max(u[i]))
            a2 = jnp.int32(_BIG_I)
            for i in range(4):
                a2 = jnp.minimum(a2, jnp.min(jnp.where(u[i] == m2, gl[i],
                                                       _BIG_I)))
            p2raw = jnp.float32(0.0)
            for i in range(4):
                p2raw = p2raw + jnp.sum(jnp.where(gl[i] == a2, e[i], 0.0))
            num1 = jnp.where(top_k >= 1, jnp.float32(1.0), 0.0)
            num2 = jnp.where(top_k >= 2, p2raw, 0.0)
            den = jnp.where(renorm != 0, num1 + num2, z)
            acc_ids = jnp.where(lane == 2 * tt, a1, acc_ids)
            acc_ids = jnp.where(lane == 2 * tt + 1, a2, acc_ids)
            acc_num = jnp.where(lane == 2 * tt, num1, acc_num)
            acc_num = jnp.where(lane == 2 * tt + 1, num2, acc_num)
            acc_den = jnp.where(lane == 2 * tt, den, acc_den)
            acc_den = jnp.where(lane == 2 * tt + 1, den, acc_den)
        idx_v[...] = acc_ids
        ws_v[...] = acc_num / acc_den
        cp_i = pltpu.async_copy(idx_v.at[pl.ds(0, 8)],
                                ids_hbm.at[pl.ds(8 * sid, 8)], sem)
        cp_w = pltpu.async_copy(ws_v.at[pl.ds(0, 8)],
                                ws_hbm.at[pl.ds(8 * sid, 8)], sem)
        cp_i.wait()
        cp_w.wait()

        plsc.subcore_barrier()

        @pl.when(sid == 0)
        def _compact():
            pltpu.sync_copy(ids_hbm, ids_all_v)
            for i in range(4):
                act_v[pl.ds(16 * i, 16)] = jnp.zeros((16,), jnp.int32)
            ones = jnp.full((16,), 1, jnp.int32)
            for i in range(8):
                idv = ids_all_v[pl.ds(16 * i, 16)]
                plsc.store_scatter(act_v, [idv], ones)
            for i in range(5):
                perm_v[pl.ds(16 * i, 16)] = jnp.zeros((16,), jnp.int32)
            base = jnp.int32(0)
            for i in range(4):
                a = act_v[pl.ds(16 * i, 16)]
                incl = plsc.cumsum(a)
                pos = incl - a + base
                plsc.store_scatter(perm_v, [pos], lane + 16 * i, mask=a > 0)
                base = base + jnp.max(incl)
            perm_v[pl.ds(64, 16)] = jnp.full((16,), 1, jnp.int32) * base
            pltpu.sync_copy(perm_v, permcnt_hbm)


def _expert_ffn(x_ref, w13_ref, w2_ref, d_ff):
    h_ = jax.lax.dot_general(x_ref[:], w13_ref[0], (((1,), (1,)), ((), ())),
                             preferred_element_type=jnp.float32)
    gate = h_[:, :d_ff]
    up = h_[:, d_ff:]
    act = gate * jax.lax.logistic(gate) * up
    return jax.lax.dot_general(act, w2_ref[0], (((1,), (1,)), ((), ())),
                               preferred_element_type=jnp.float32)


def _ffn_kernel(pc_ref, ids_ref, ws_ref, x_ref, w13_ref, w2_ref,
                out_ref, *, d_ff, n_e):
    i = pl.program_id(0)
    cnt = pc_ref[n_e]

    @pl.when(i == 0)
    def _init():
        out_ref[:] = jnp.zeros_like(out_ref)

    @pl.when(i < cnt)
    def _compute():
        e = pc_ref[jnp.minimum(i, jnp.maximum(cnt - 1, 0))]
        o = _expert_ffn(x_ref, w13_ref, w2_ref, d_ff)
        c = (jnp.where(ids_ref[:, 0:1] == e, ws_ref[:, 0:1], 0.0)
             + jnp.where(ids_ref[:, 1:2] == e, ws_ref[:, 1:2], 0.0))
        out_ref[:] += c * o


def kernel(x, router_logits, w13_weight, w2_weight, top_k, renormalize):
    t, h = x.shape
    n_e = w13_weight.shape[0]
    d_ff = w13_weight.shape[1] // 2
    lane16 = jnp.arange(16, dtype=jnp.int32)
    params16 = (jnp.where(lane16 == 0, jnp.asarray(top_k, jnp.int32), 0)
                + jnp.where(lane16 == 1, jnp.asarray(renormalize, jnp.int32),
                            0))

    mesh = plsc.VectorSubcoreMesh(core_axis_name="c", subcore_axis_name="s",
                                  num_cores=1)
    routing = functools.partial(
        pl.kernel,
        mesh=mesh,
        compiler_params=pltpu.CompilerParams(needs_layout_passes=False),
        out_type=[
            jax.ShapeDtypeStruct((2 * t,), jnp.int32),
            jax.ShapeDtypeStruct((2 * t,), jnp.float32),
            jax.ShapeDtypeStruct((n_e + 16,), jnp.int32),
        ],
        scratch_types=[
            pltpu.VMEM((4, n_e), jnp.float32),
            pltpu.VMEM((16,), jnp.int32),
            pltpu.VMEM((16,), jnp.float32),
            pltpu.VMEM((16,), jnp.int32),
            pltpu.VMEM((2 * t,), jnp.int32),
            pltpu.VMEM((n_e,), jnp.int32),
            pltpu.VMEM((n_e + 16,), jnp.int32),
            pltpu.SemaphoreType.DMA,
        ],
    )(_sc_routing)
    ids_flat, ws_flat, permcnt = routing(router_logits, params16)
    ids = ids_flat.reshape(t, 2)
    ws = ws_flat.reshape(t, 2)

    def _w_map(i, pc_ref):
        return (pc_ref[jnp.minimum(i, jnp.maximum(pc_ref[n_e] - 1, 0))], 0, 0)

    body = functools.partial(_ffn_kernel, d_ff=d_ff, n_e=n_e)
    grid_spec = pltpu.PrefetchScalarGridSpec(
        num_scalar_prefetch=1,
        grid=(n_e,),
        in_specs=[
            pl.BlockSpec((t, 2), lambda i, p: (0, 0)),
            pl.BlockSpec((t, 2), lambda i, p: (0, 0)),
            pl.BlockSpec((t, h), lambda i, p: (0, 0)),
            pl.BlockSpec((1, 2 * d_ff, h), _w_map),
            pl.BlockSpec((1, h, d_ff), _w_map),
        ],
        out_specs=pl.BlockSpec((t, h), lambda i, p: (0, 0)),
    )
    out = pl.pallas_call(
        body,
        grid_spec=grid_spec,
        out_shape=jax.ShapeDtypeStruct((t, h), jnp.float32),
        compiler_params=pltpu.CompilerParams(
            dimension_semantics=("arbitrary",),
        ),
    )(permcnt, ids, ws, x, w13_weight, w2_weight)
    return out.astype(x.dtype)
